# baseline (speedup 1.0000x reference)
import jax
import jax.numpy as jnp
from jax.experimental import pallas as pl
from jax.experimental.pallas import tpu as pltpu

_N = 50000
_HDIM = 256
_NUMHEADS = 2
_OUTDIM = 256
_NUM_GRAPHS = 256

_T = 2048
_NSTREAM = 2
_CHUNK = _T * _NSTREAM
_NPAD = ((_N + _CHUNK - 1) // _CHUNK) * _CHUNK
_NSTEP = _NPAD // _CHUNK
_HHALF = _HDIM // _NUMHEADS
_NEG = -1e30


def _tile_stats(blk, ids_row, attT):
    sT = jax.lax.dot_general(attT, blk, (((1,), (1,)), ((), ())),
                             preferred_element_type=jnp.float32)
    seg_iota = jax.lax.broadcasted_iota(jnp.int32, (_NUM_GRAPHS, _T), 0)
    pt_bool = seg_iota == ids_row
    parts = []
    for h in range(_NUMHEADS):
        m = jnp.where(pt_bool, sT[h : h + 1, :], _NEG)
        parts.append(jnp.max(m, axis=1)[None, :])
    return sT, pt_bool, jnp.concatenate(parts, axis=0)


def _tile_acc(blk, sT, pt_bool, newmax):
    pt_f32 = pt_bool.astype(jnp.float32)
    nmT = jax.lax.dot_general(newmax, pt_f32, (((1,), (0,)), ((), ())),
                              preferred_element_type=jnp.float32)
    exT = jnp.exp(sT - nmT)
    dden = jax.lax.dot_general(exT, pt_f32, (((1,), (1,)), ((), ())),
                               preferred_element_type=jnp.float32)
    dnum = []
    for h in range(_NUMHEADS):
        ptw = pt_f32 * exT[h : h + 1, :]
        lo, hi = h * _HHALF, (h + 1) * _HHALF
        dnum.append(jax.lax.dot_general(ptw, blk[:, lo:hi],
                                        (((1,), (0,)), ((), ())),
                                        preferred_element_type=jnp.float32))
    return dden, dnum


def _body(sa_ref, sb_ref, ida_ref, idb_ref, attT_ref, w_ref, b_ref, out_ref,
          maxacc_ref, numer_ref, den_ref):
    i = pl.program_id(0)

    @pl.when(i == 0)
    def _init():
        maxacc_ref[...] = jnp.full((_NUMHEADS, _NUM_GRAPHS), _NEG, jnp.float32)
        numer_ref[...] = jnp.zeros((_NUM_GRAPHS, _HDIM), jnp.float32)
        den_ref[...] = jnp.zeros((_NUMHEADS, _NUM_GRAPHS), jnp.float32)

    blk_a = sa_ref[...]
    blk_b = sb_ref[...]
    attT = attT_ref[...]
    sT_a, pt_a, tmax_a = _tile_stats(blk_a, ida_ref[0], attT)
    sT_b, pt_b, tmax_b = _tile_stats(blk_b, idb_ref[0], attT)

    newmax = jnp.maximum(maxacc_ref[...], jnp.maximum(tmax_a, tmax_b))
    alpha = jnp.exp(maxacc_ref[...] - newmax)
    maxacc_ref[...] = newmax

    dden_a, dnum_a = _tile_acc(blk_a, sT_a, pt_a, newmax)
    dden_b, dnum_b = _tile_acc(blk_b, sT_b, pt_b, newmax)
    den_ref[...] = den_ref[...] * alpha + dden_a + dden_b

    r = jax.lax.broadcasted_iota(jnp.int32, (_NUM_GRAPHS, _NUM_GRAPHS), 0)
    c = jax.lax.broadcasted_iota(jnp.int32, (_NUM_GRAPHS, _NUM_GRAPHS), 1)
    eye = (r == c).astype(jnp.float32)
    acol = jax.lax.dot_general(eye, alpha, (((1,), (1,)), ((), ())),
                               preferred_element_type=jnp.float32)
    for h in range(_NUMHEADS):
        lo, hi = h * _HHALF, (h + 1) * _HHALF
        numer_ref[:, lo:hi] = (numer_ref[:, lo:hi] * acol[:, h : h + 1]
                               + dnum_a[h] + dnum_b[h])

    @pl.when(i == _NSTEP - 1)
    def _finish():
        den = den_ref[...]
        dinv = jnp.where(den > 0, 1.0 / den, 0.0)
        dcol = jax.lax.dot_general(eye, dinv, (((1,), (1,)), ((), ())),
                                   preferred_element_type=jnp.float32)
        lane = jax.lax.broadcasted_iota(jnp.int32, (_NUM_GRAPHS, _HDIM), 1)
        scale = jnp.where(lane < _HHALF, dcol[:, 0:1], dcol[:, 1:2])
        attn = numer_ref[...] * scale
        out_ref[...] = jax.lax.dot_general(attn, w_ref[...],
                                           (((1,), (1,)), ((), ())),
                                           preferred_element_type=jnp.float32
                                           ) + b_ref[...]


@jax.jit
def kernel(states, segment_ids, att_vecs, W, b):
    pad = _NPAD - _N
    states_p = jnp.pad(states, ((0, pad), (0, 0)))
    ids3 = jnp.pad(segment_ids.astype(jnp.int32), (0, pad),
                   constant_values=_NUM_GRAPHS).reshape(2 * _NSTEP, 1, _T)
    attT = att_vecs.T
    b2d = b.reshape(1, _OUTDIM)

    ret = pl.pallas_call(
        _body,
        grid=(_NSTEP,),
        in_specs=[
            pl.BlockSpec((_T, _HDIM), lambda i: (i, 0)),
            pl.BlockSpec((_T, _HDIM), lambda i: (i + _NSTEP, 0)),
            pl.BlockSpec((1, 1, _T), lambda i: (i, 0, 0)),
            pl.BlockSpec((1, 1, _T), lambda i: (i + _NSTEP, 0, 0)),
            pl.BlockSpec((_NUMHEADS, _HDIM), lambda i: (0, 0)),
            pl.BlockSpec((_OUTDIM, _HDIM), lambda i: (0, 0)),
            pl.BlockSpec((1, _OUTDIM), lambda i: (0, 0)),
        ],
        out_specs=pl.BlockSpec((_NUM_GRAPHS, _OUTDIM), lambda i: (0, 0)),
        out_shape=jax.ShapeDtypeStruct((_NUM_GRAPHS, _OUTDIM), jnp.float32),
        scratch_shapes=[
            pltpu.VMEM((_NUMHEADS, _NUM_GRAPHS), jnp.float32),
            pltpu.VMEM((_NUM_GRAPHS, _HDIM), jnp.float32),
            pltpu.VMEM((_NUMHEADS, _NUM_GRAPHS), jnp.float32),
        ],
    )(states_p, states_p, ids3, ids3, attT, W, b2d)
    return ret


# speedup vs baseline: 1.0138x; 1.0138x over previous
import jax
import jax.numpy as jnp
from jax.experimental import pallas as pl
from jax.experimental.pallas import tpu as pltpu

_N = 50000
_HDIM = 256
_NUMHEADS = 2
_OUTDIM = 256
_NUM_GRAPHS = 256

_T = 2048
_NSTREAM = 2
_CHUNK = _T * _NSTREAM
_NPAD = ((_N + _CHUNK - 1) // _CHUNK) * _CHUNK
_NSTEP = _NPAD // _CHUNK
_HHALF = _HDIM // _NUMHEADS
_NEG = -1e30


def _tile_stats(blk_bf, ids_row, attT_bf):
    sT = jax.lax.dot_general(attT_bf, blk_bf, (((1,), (1,)), ((), ())),
                             preferred_element_type=jnp.float32)
    seg_iota = jax.lax.broadcasted_iota(jnp.int32, (_NUM_GRAPHS, _T), 0)
    pt_bool = seg_iota == ids_row
    parts = []
    for h in range(_NUMHEADS):
        m = jnp.where(pt_bool, sT[h : h + 1, :], _NEG)
        parts.append(jnp.max(m, axis=1)[None, :])
    tilemax = jnp.concatenate(parts, axis=0)
    tilemax = tilemax.astype(jnp.bfloat16).astype(jnp.float32)
    return sT, pt_bool, tilemax


def _tile_acc(blk_bf, sT, pt_bool, newmax_bf):
    pt_bf = pt_bool.astype(jnp.bfloat16)
    nmT = jax.lax.dot_general(newmax_bf, pt_bf, (((1,), (0,)), ((), ())),
                              preferred_element_type=jnp.float32)
    exT_bf = jnp.exp(sT - nmT).astype(jnp.bfloat16)
    dden = jax.lax.dot_general(exT_bf, pt_bf, (((1,), (1,)), ((), ())),
                               preferred_element_type=jnp.float32)
    dnum = []
    for h in range(_NUMHEADS):
        ptw = pt_bf * exT_bf[h : h + 1, :]
        lo, hi = h * _HHALF, (h + 1) * _HHALF
        dnum.append(jax.lax.dot_general(ptw, blk_bf[:, lo:hi],
                                        (((1,), (0,)), ((), ())),
                                        preferred_element_type=jnp.float32))
    return dden, dnum


def _body(sa_ref, sb_ref, ida_ref, idb_ref, attT_ref, w_ref, b_ref, out_ref,
          maxacc_ref, numer_ref, den_ref):
    i = pl.program_id(0)

    @pl.when(i == 0)
    def _init():
        maxacc_ref[...] = jnp.full((_NUMHEADS, _NUM_GRAPHS), _NEG, jnp.float32)
        numer_ref[...] = jnp.zeros((_NUM_GRAPHS, _HDIM), jnp.float32)
        den_ref[...] = jnp.zeros((_NUMHEADS, _NUM_GRAPHS), jnp.float32)

    blk_a = sa_ref[...].astype(jnp.bfloat16)
    blk_b = sb_ref[...].astype(jnp.bfloat16)
    attT_bf = attT_ref[...].astype(jnp.bfloat16)
    sT_a, pt_a, tmax_a = _tile_stats(blk_a, ida_ref[0], attT_bf)
    sT_b, pt_b, tmax_b = _tile_stats(blk_b, idb_ref[0], attT_bf)

    newmax = jnp.maximum(maxacc_ref[...], jnp.maximum(tmax_a, tmax_b))
    alpha = jnp.exp(maxacc_ref[...] - newmax)
    maxacc_ref[...] = newmax
    newmax_bf = newmax.astype(jnp.bfloat16)

    dden_a, dnum_a = _tile_acc(blk_a, sT_a, pt_a, newmax_bf)
    dden_b, dnum_b = _tile_acc(blk_b, sT_b, pt_b, newmax_bf)
    den_ref[...] = den_ref[...] * alpha + dden_a + dden_b

    r = jax.lax.broadcasted_iota(jnp.int32, (_NUM_GRAPHS, _NUM_GRAPHS), 0)
    c = jax.lax.broadcasted_iota(jnp.int32, (_NUM_GRAPHS, _NUM_GRAPHS), 1)
    eye = (r == c).astype(jnp.float32)
    acol = jax.lax.dot_general(eye, alpha, (((1,), (1,)), ((), ())),
                               preferred_element_type=jnp.float32)
    for h in range(_NUMHEADS):
        lo, hi = h * _HHALF, (h + 1) * _HHALF
        numer_ref[:, lo:hi] = (numer_ref[:, lo:hi] * acol[:, h : h + 1]
                               + dnum_a[h] + dnum_b[h])

    @pl.when(i == _NSTEP - 1)
    def _finish():
        den = den_ref[...]
        dinv = jnp.where(den > 0, 1.0 / den, 0.0)
        dcol = jax.lax.dot_general(eye, dinv, (((1,), (1,)), ((), ())),
                                   preferred_element_type=jnp.float32)
        lane = jax.lax.broadcasted_iota(jnp.int32, (_NUM_GRAPHS, _HDIM), 1)
        scale = jnp.where(lane < _HHALF, dcol[:, 0:1], dcol[:, 1:2])
        attn = numer_ref[...] * scale
        out_ref[...] = jax.lax.dot_general(attn, w_ref[...],
                                           (((1,), (1,)), ((), ())),
                                           preferred_element_type=jnp.float32
                                           ) + b_ref[...]


@jax.jit
def kernel(states, segment_ids, att_vecs, W, b):
    pad = _NPAD - _N
    states_p = jnp.pad(states, ((0, pad), (0, 0)))
    ids3 = jnp.pad(segment_ids.astype(jnp.int32), (0, pad),
                   constant_values=_NUM_GRAPHS).reshape(2 * _NSTEP, 1, _T)
    attT = att_vecs.T
    b2d = b.reshape(1, _OUTDIM)

    ret = pl.pallas_call(
        _body,
        grid=(_NSTEP,),
        in_specs=[
            pl.BlockSpec((_T, _HDIM), lambda i: (i, 0)),
            pl.BlockSpec((_T, _HDIM), lambda i: (i + _NSTEP, 0)),
            pl.BlockSpec((1, 1, _T), lambda i: (i, 0, 0)),
            pl.BlockSpec((1, 1, _T), lambda i: (i + _NSTEP, 0, 0)),
            pl.BlockSpec((_NUMHEADS, _HDIM), lambda i: (0, 0)),
            pl.BlockSpec((_OUTDIM, _HDIM), lambda i: (0, 0)),
            pl.BlockSpec((1, _OUTDIM), lambda i: (0, 0)),
        ],
        out_specs=pl.BlockSpec((_NUM_GRAPHS, _OUTDIM), lambda i: (0, 0)),
        out_shape=jax.ShapeDtypeStruct((_NUM_GRAPHS, _OUTDIM), jnp.float32),
        scratch_shapes=[
            pltpu.VMEM((_NUMHEADS, _NUM_GRAPHS), jnp.float32),
            pltpu.VMEM((_NUM_GRAPHS, _HDIM), jnp.float32),
            pltpu.VMEM((_NUMHEADS, _NUM_GRAPHS), jnp.float32),
        ],
    )(states_p, states_p, ids3, ids3, attT, W, b2d)
    return ret
